# Initial kernel scaffold; baseline (speedup 1.0000x reference)
import jax
import jax.numpy as jnp
from jax.experimental import pallas as pl


def kernel(x, conv_w, conv_b, bn_gamma, bn_beta, gate_w, gate_b):
    raise NotImplementedError("write your pallas kernel here")


# speedup vs baseline: 12.0811x; 12.0811x over previous
import jax
import jax.numpy as jnp
from jax.experimental import pallas as pl
from jax.experimental.pallas import tpu as pltpu


def _pool_stats_kernel(xq_ref, xp_ref, mx_ref, s_ref):
    b = pl.program_id(0)
    xq = xq_ref[0]
    p = xq.shape[1]
    xp = jnp.sum(xq, axis=1) * (1.0 / p)
    xp_ref[0] = xp
    mx_ref[0] = jnp.mean(xp, axis=-1, keepdims=True)
    prod = jax.lax.dot_general(xp, xp, (((1,), (1,)), ((), ())),
                               preferred_element_type=jnp.float32)

    @pl.when(b == 0)
    def _():
        s_ref[...] = prod

    @pl.when(b != 0)
    def _():
        s_ref[...] += prod


def _gate_fold_kernel(n_experts, n_count,
                      mx_ref, s_ref, gw_ref, gb_ref, cw_ref, cb_ref,
                      gam_ref, bet_ref,
                      wf_ref, bf_ref, eidx_ref, ew_ref):
    mx = mx_ref[...]
    bsz, _ = mx.shape
    n_logits = gw_ref.shape[0]
    logits = jax.lax.dot_general(mx, gw_ref[...], (((1,), (1,)), ((), ())),
                                 preferred_element_type=jnp.float32)
    logits = logits + gb_ref[...]
    z = logits - jnp.max(logits, axis=-1, keepdims=True)
    ez = jnp.exp(z)
    sm = ez / jnp.sum(ez, axis=-1, keepdims=True)
    cols = jax.lax.broadcasted_iota(jnp.int32, sm.shape, 1)
    v1 = jnp.max(sm, axis=-1, keepdims=True)
    a1 = jnp.min(jnp.where(sm == v1, cols, n_logits), axis=-1, keepdims=True)
    sm2 = jnp.where(cols == a1, -1.0, sm)
    v2 = jnp.max(sm2, axis=-1, keepdims=True)
    a2 = jnp.min(jnp.where(sm2 == v2, cols, n_logits), axis=-1, keepdims=True)
    den = v1 + v2
    w1 = jnp.where(a1 < n_experts, v1 / den, 0.0)
    w2 = jnp.where(a2 < n_experts, v2 / den, 0.0)
    e1 = jnp.minimum(a1, n_experts - 1)
    e2 = jnp.minimum(a2, n_experts - 1)
    eidx_ref[...] = jnp.concatenate([e1, e2], axis=1)
    ew_ref[...] = jnp.concatenate([w1, w2], axis=1)

    mean_all = jnp.mean(mx, axis=0, keepdims=True)
    outer = jax.lax.dot_general(mean_all, mean_all, (((0,), (0,)), ((), ())),
                                preferred_element_type=jnp.float32)
    cov = s_ref[...] * (1.0 / n_count) - outer
    cw = cw_ref[...]
    ws = jax.lax.dot_general(cw, cov, (((1,), (0,)), ((), ())),
                             preferred_element_type=jnp.float32)
    var = jnp.sum(ws * cw, axis=-1, keepdims=True)
    mu_x = jax.lax.dot_general(cw, mean_all, (((1,), (1,)), ((), ())),
                               preferred_element_type=jnp.float32)
    inv = gam_ref[...] * jax.lax.rsqrt(var + 1e-5)
    wf_ref[...] = cw * inv
    bf_ref[...] = -mu_x * inv + bet_ref[...]


def _moe_kernel(eidx_ref, ew_ref, wf_ref, bf_ref, xp_ref, out_ref):
    b = pl.program_id(0)
    k = pl.program_id(1)
    w = ew_ref[b, k]
    z = jax.lax.dot_general(wf_ref[0], xp_ref[0], (((1,), (0,)), ((), ())),
                            preferred_element_type=jnp.float32)
    y = jnp.maximum(z + bf_ref[0], 0.0) * w

    @pl.when(k == 0)
    def _():
        out_ref[0] = y

    @pl.when(k != 0)
    def _():
        out_ref[0] += y


def kernel(x, conv_w, conv_b, bn_gamma, bn_beta, gate_w, gate_b):
    B, NB, T = x.shape
    E, C, _ = conv_w.shape
    P = 4
    K = 2
    TP = T // P
    N = B * TP

    f32 = jnp.float32
    xq = x.reshape(B, NB, TP, P).swapaxes(2, 3)

    xp, mx3, s = pl.pallas_call(
        _pool_stats_kernel,
        grid=(B,),
        in_specs=[pl.BlockSpec((1, NB, P, TP), lambda b: (b, 0, 0, 0))],
        out_specs=[pl.BlockSpec((1, NB, TP), lambda b: (b, 0, 0)),
                   pl.BlockSpec((1, NB, 1), lambda b: (b, 0, 0)),
                   pl.BlockSpec((NB, NB), lambda b: (0, 0))],
        out_shape=[jax.ShapeDtypeStruct((B, NB, TP), f32),
                   jax.ShapeDtypeStruct((B, NB, 1), f32),
                   jax.ShapeDtypeStruct((NB, NB), f32)],
    )(xq)
    mx = mx3.reshape(B, NB)

    import functools
    gate_fold = functools.partial(_gate_fold_kernel, E, N)
    wf_flat, bf_flat, eidx, ew = pl.pallas_call(
        gate_fold,
        out_shape=[jax.ShapeDtypeStruct((E * C, NB), f32),
                   jax.ShapeDtypeStruct((E * C, 1), f32),
                   jax.ShapeDtypeStruct((B, K), jnp.int32),
                   jax.ShapeDtypeStruct((B, K), f32)],
    )(mx, s, gate_w, gate_b.reshape(1, C),
      conv_w.reshape(E * C, NB), conv_b.reshape(E * C, 1),
      bn_gamma.reshape(E * C, 1), bn_beta.reshape(E * C, 1))

    wf = wf_flat.reshape(E, C, NB)
    bf = bf_flat.reshape(E, C, 1)

    out = pl.pallas_call(
        _moe_kernel,
        grid_spec=pltpu.PrefetchScalarGridSpec(
            num_scalar_prefetch=2,
            grid=(B, K),
            in_specs=[
                pl.BlockSpec((1, C, NB), lambda b, k, ei, w: (ei[b, k], 0, 0)),
                pl.BlockSpec((1, C, 1), lambda b, k, ei, w: (ei[b, k], 0, 0)),
                pl.BlockSpec((1, NB, TP), lambda b, k, ei, w: (b, 0, 0)),
            ],
            out_specs=pl.BlockSpec((1, C, TP), lambda b, k, ei, w: (b, 0, 0)),
        ),
        out_shape=jax.ShapeDtypeStruct((B, C, TP), f32),
    )(eidx, ew, wf, bf, xp)
    return out
